# baseline (speedup 1.0000x reference)
import functools

import jax
import jax.numpy as jnp
from jax import lax
from jax.experimental import pallas as pl
from jax.experimental.pallas import tpu as pltpu
from jax.experimental.pallas import tpu_sc as plsc

_NC = 2
_NS = 16
_NW = _NC * _NS
_H = 8
_HD = 16
_BSLOT = 16


def _qkv_body(x_ref, wq_ref, bq_ref, wk_ref, bk_ref, wv_ref, bv_ref,
              q_ref, k_ref, v_ref):
    xb = x_ref[...]
    q_ref[...] = jnp.dot(xb, wq_ref[...], preferred_element_type=jnp.float32) + bq_ref[...]
    k_ref[...] = jnp.dot(xb, wk_ref[...], preferred_element_type=jnp.float32) + bk_ref[...]
    v_ref[...] = jnp.dot(xb, wv_ref[...], preferred_element_type=jnp.float32) + bv_ref[...]


def _qkv(x, WQ, bQ, WK, bK, WV, bV, block_rows):
    n, d = x.shape
    grid = (n // block_rows,)
    row_spec = pl.BlockSpec((block_rows, d), lambda i: (i, 0))
    w_spec = pl.BlockSpec((d, d), lambda i: (0, 0))
    b_spec = pl.BlockSpec((1, d), lambda i: (0, 0))
    out = jax.ShapeDtypeStruct((n, d), jnp.float32)
    return pl.pallas_call(
        _qkv_body,
        grid=grid,
        in_specs=[row_spec, w_spec, b_spec, w_spec, b_spec, w_spec, b_spec],
        out_specs=[row_spec, row_spec, row_spec],
        out_shape=[out, out, out],
    )(x, WQ, bQ.reshape(1, d), WK, bK.reshape(1, d), WV, bV.reshape(1, d))


def _gather_body(n_edges, chunk, qx, kx, vx, src, dst,
                 qs_out, kd_out, vd_out,
                 src_v, dst_v, qb, kb, vb, sem_q, sem_k, sem_v):
    c = lax.axis_index("c")
    s = lax.axis_index("s")
    w = c * _NS + s
    epw = n_edges // _NW
    n_chunks = epw // chunk

    def _chunk_body(ci, _):
        base = w * epw + ci * chunk
        pltpu.sync_copy(src.at[pl.ds(base, chunk)], src_v)
        pltpu.sync_copy(dst.at[pl.ds(base, chunk)], dst_v)
        cp_q = pltpu.async_copy(qx.at[src_v], qb, sem_q)
        cp_k = pltpu.async_copy(kx.at[dst_v], kb, sem_k)
        cp_v = pltpu.async_copy(vx.at[dst_v], vb, sem_v)
        cp_q.wait()
        cp_k.wait()
        cp_v.wait()
        pltpu.sync_copy(qb, qs_out.at[pl.ds(base, chunk)])
        pltpu.sync_copy(kb, kd_out.at[pl.ds(base, chunk)])
        pltpu.sync_copy(vb, vd_out.at[pl.ds(base, chunk)])
        return 0

    lax.fori_loop(0, n_chunks, _chunk_body, 0)


def _gather_phase(qx, kx, vx, src, dst, chunk=200):
    n, d = qx.shape
    e = src.shape[0]
    mesh = plsc.VectorSubcoreMesh(core_axis_name="c", subcore_axis_name="s")
    out = jax.ShapeDtypeStruct((e, d), jnp.float32)
    kfn = pl.kernel(
        functools.partial(_gather_body, e, chunk),
        out_type=(out, out, out),
        mesh=mesh,
        scratch_types=(
            pltpu.VMEM((chunk,), jnp.int32),
            pltpu.VMEM((chunk,), jnp.int32),
            pltpu.VMEM((chunk, d), jnp.float32),
            pltpu.VMEM((chunk, d), jnp.float32),
            pltpu.VMEM((chunk, d), jnp.float32),
            pltpu.SemaphoreType.DMA,
            pltpu.SemaphoreType.DMA,
            pltpu.SemaphoreType.DMA,
        ),
    )
    return kfn(qx, kx, vx, src, dst)


def _edge_math_body(qs_ref, kd_ref, vd_ref, w_ref, den_ref):
    p = qs_ref[...] * kd_ref[...]
    d = p.shape[-1]
    rows = lax.broadcasted_iota(jnp.int32, (d, d), 0)
    cols = lax.broadcasted_iota(jnp.int32, (d, d), 1)
    rep = (rows // _HD == cols // _HD).astype(jnp.float32)
    logits = jnp.dot(p, rep, preferred_element_type=jnp.float32)
    wrep = jnp.exp(logits * jnp.float32(0.25))
    w_ref[...] = wrep * vd_ref[...]
    rows2 = lax.broadcasted_iota(jnp.int32, (d, _HD), 0)
    cols2 = lax.broadcasted_iota(jnp.int32, (d, _HD), 1)
    sel = ((rows2 // _HD == cols2) & (cols2 < _H)).astype(jnp.float32) * jnp.float32(1.0 / _HD)
    den_ref[...] = jnp.dot(wrep, sel, preferred_element_type=jnp.float32)


def _edge_math(qs, kd, vd, block_rows):
    e, d = qs.shape
    grid = (e // block_rows,)
    row_spec = pl.BlockSpec((block_rows, d), lambda i: (i, 0))
    den_spec = pl.BlockSpec((block_rows, _HD), lambda i: (i, 0))
    return pl.pallas_call(
        _edge_math_body,
        grid=grid,
        in_specs=[row_spec, row_spec, row_spec],
        out_specs=[row_spec, den_spec],
        out_shape=[jax.ShapeDtypeStruct((e, d), jnp.float32),
                   jax.ShapeDtypeStruct((e, _HD), jnp.float32)],
    )(qs, kd, vd)


def _seg_body(n_nodes, n_edges, chunk, wrows, drows, srcrep, tgt,
              num_out, den_out, bnum_out, bden_out, bidx_out,
              tgt_v, wb, db, srb, sbuf, sdbuf, bbuf, bdbuf, bibuf,
              sem_w, sem_d):
    c = lax.axis_index("c")
    s = lax.axis_index("s")
    w = c * _NS + s
    epw = n_edges // _NW
    n_chunks = epw // chunk
    npad = n_nodes + 8
    lane = jnp.arange(_HD, dtype=jnp.int32)
    zero_row = jnp.zeros((_HD,), jnp.float32)

    def _zero_bufs(i, _):
        for j in range(_H):
            wb[i, pl.ds(j * _HD, _HD)] = zero_row
            sdbuf[i, pl.ds(j * _HD, _HD)] = zero_row
        db[i, :] = zero_row
        return 0

    lax.fori_loop(0, chunk, _zero_bufs, 0)

    rz = (npad // (8 * _NS)) * 8
    rem = npad - rz * _NS
    row0 = c * npad + s * rz

    def _zero_out(base, nrows):
        off = 0
        while off < nrows:
            step = min(chunk, nrows - off)
            pltpu.sync_copy(wb.at[pl.ds(0, step)], num_out.at[pl.ds(base + off, step)])
            pltpu.sync_copy(sdbuf.at[pl.ds(0, step)], den_out.at[pl.ds(base + off, step)])
            off += step

    _zero_out(row0, rz)

    @pl.when(s == _NS - 1)
    def _():
        _zero_out(c * npad + rz * _NS, rem)

    plsc.subcore_barrier()

    def _chunk_body(ci, carry):
        curv, accs, dacc = carry
        base = w * epw + ci * chunk
        pltpu.sync_copy(tgt.at[pl.ds(base, chunk)], tgt_v)
        cp_s = pltpu.async_copy(srcrep.at[pl.ds(base, chunk)], srb, sem_w)
        cp_w = pltpu.async_copy(wrows.at[pl.ds(base, chunk)], wb, sem_d)
        cp_s.wait()
        cp_w.wait()
        cp_d = pltpu.async_copy(drows.at[pl.ds(base, chunk)], db, sem_d)
        cp_d.wait()

        def _edge(e, ecarry):
            curv, accs, dacc = ecarry
            sv = srb[e, :]
            ch = sv != curv
            new_accs = []
            for h in range(_H):
                wv = wb[e, pl.ds(h * _HD, _HD)]
                na = jnp.where(ch, wv, accs[h] + wv)
                sbuf[e, pl.ds(h * _HD, _HD)] = na
                new_accs.append(na)
            dv = db[e, :]
            dacc = jnp.where(ch, dv, dacc + dv)
            sdbuf[e, pl.ds(0, _HD)] = dacc
            return sv, tuple(new_accs), dacc

        curv, accs, dacc = lax.fori_loop(0, chunk, _edge, (curv, accs, dacc))

        cp_n = pltpu.async_copy(sbuf, num_out.at[tgt_v], sem_w)
        cp_dn = pltpu.async_copy(sdbuf, den_out.at[tgt_v], sem_d)
        cp_n.wait()
        cp_dn.wait()
        return curv, accs, dacc

    acc0 = tuple(jnp.zeros((_HD,), jnp.float32) for _ in range(_H))
    curv, accs, dacc = lax.fori_loop(
        0, n_chunks, _chunk_body,
        (jnp.full((_HD,), -1, jnp.int32), acc0, jnp.zeros((_HD,), jnp.float32)))

    for h in range(_H):
        bbuf[1, pl.ds(h * _HD, _HD)] = accs[h]
        bbuf[0, pl.ds(h * _HD, _HD)] = zero_row
    bdbuf[1, :] = dacc
    bdbuf[0, :] = zero_row
    bibuf[...] = jnp.where(lane == 1, curv, jnp.full((_HD,), -1, jnp.int32))

    pltpu.sync_copy(bbuf, bnum_out.at[pl.ds(w * _BSLOT, _BSLOT)])
    pltpu.sync_copy(bdbuf, bden_out.at[pl.ds(w * _BSLOT, _BSLOT)])
    pltpu.sync_copy(bibuf, bidx_out.at[pl.ds(w * _BSLOT, _BSLOT)])


def _seg_phase(wrows, drows, srcrep, tgt, n_nodes, chunk=80):
    e, d = wrows.shape
    npad = n_nodes + 8
    mesh = plsc.VectorSubcoreMesh(core_axis_name="c", subcore_axis_name="s")
    kfn = pl.kernel(
        functools.partial(_seg_body, n_nodes, e, chunk),
        out_type=(
            jax.ShapeDtypeStruct((_NC * npad, d), jnp.float32),
            jax.ShapeDtypeStruct((_NC * npad, d), jnp.float32),
            jax.ShapeDtypeStruct((_NW * _BSLOT, d), jnp.float32),
            jax.ShapeDtypeStruct((_NW * _BSLOT, _HD), jnp.float32),
            jax.ShapeDtypeStruct((_NW * _BSLOT,), jnp.int32),
        ),
        mesh=mesh,
        scratch_types=(
            pltpu.VMEM((chunk,), jnp.int32),
            pltpu.VMEM((chunk, d), jnp.float32),
            pltpu.VMEM((chunk, _HD), jnp.float32),
            pltpu.VMEM((chunk, _HD), jnp.int32),
            pltpu.VMEM((chunk, d), jnp.float32),
            pltpu.VMEM((chunk, d), jnp.float32),
            pltpu.VMEM((_BSLOT, d), jnp.float32),
            pltpu.VMEM((_BSLOT, _HD), jnp.float32),
            pltpu.VMEM((_BSLOT,), jnp.int32),
            pltpu.SemaphoreType.DMA,
            pltpu.SemaphoreType.DMA,
        ),
    )
    return kfn(wrows, drows, srcrep, tgt)


def _merge_body(num_ref, den_ref, bnum_ref, bden_ref, bidx_ref,
                onum_ref, oden_ref):
    onum_ref[...] = num_ref[0] + num_ref[1]
    oden_ref[...] = den_ref[0] + den_ref[1]

    def _apply(i, _):
        idx = bidx_ref[i]

        @pl.when(idx >= 0)
        def _():
            onum_ref[pl.ds(idx, 1), :] = onum_ref[pl.ds(idx, 1), :] + bnum_ref[pl.ds(i, 1), :]
            oden_ref[pl.ds(idx, 1), :] = oden_ref[pl.ds(idx, 1), :] + bden_ref[pl.ds(i, 1), :]

        return 0

    lax.fori_loop(0, _NW * _BSLOT, _apply, 0)


def _merge_phase(num2, den2, bnum, bden, bidx, n_nodes):
    d = num2.shape[-1]
    nb = _NW * _BSLOT
    return pl.pallas_call(
        _merge_body,
        in_specs=[
            pl.BlockSpec((_NC, n_nodes, d), lambda: (0, 0, 0)),
            pl.BlockSpec((_NC, n_nodes, _HD), lambda: (0, 0, 0)),
            pl.BlockSpec((nb, d), lambda: (0, 0)),
            pl.BlockSpec((nb, _HD), lambda: (0, 0)),
            pl.BlockSpec(memory_space=pltpu.SMEM),
        ],
        out_specs=[pl.BlockSpec((n_nodes, d), lambda: (0, 0)),
                   pl.BlockSpec((n_nodes, _HD), lambda: (0, 0))],
        out_shape=[jax.ShapeDtypeStruct((n_nodes, d), jnp.float32),
                   jax.ShapeDtypeStruct((n_nodes, _HD), jnp.float32)],
    )(num2, den2, bnum, bden, bidx)


def _out_body(num_ref, den_ref, wout_ref, bout_ref,
              ln1w_ref, ln1b_ref, ln2w_ref, ln2b_ref, o_ref):
    nb = num_ref[...]
    db = den_ref[...]
    d = nb.shape[-1]
    rows = lax.broadcasted_iota(jnp.int32, (_HD, d), 0)
    cols = lax.broadcasted_iota(jnp.int32, (_HD, d), 1)
    rep_m = (cols // _HD == rows).astype(jnp.float32)
    rep = jnp.dot(db, rep_m, preferred_element_type=jnp.float32)
    attn = nb / (rep + jnp.float32(1e-16))
    mu = jnp.mean(attn, axis=1, keepdims=True)
    xc = attn - mu
    var = jnp.mean(xc * xc, axis=1, keepdims=True)
    ln1 = xc / jnp.sqrt(var + jnp.float32(1e-5)) * ln1w_ref[...] + ln1b_ref[...]
    o = jnp.dot(ln1, wout_ref[...], preferred_element_type=jnp.float32) + bout_ref[...]
    mu2 = jnp.mean(o, axis=1, keepdims=True)
    oc = o - mu2
    var2 = jnp.mean(oc * oc, axis=1, keepdims=True)
    o_ref[...] = oc / jnp.sqrt(var2 + jnp.float32(1e-5)) * ln2w_ref[...] + ln2b_ref[...]


def _out_phase(num, den, Wout, bout, ln1_w, ln1_b, ln2_w, ln2_b, block_rows):
    n, d = num.shape
    grid = (n // block_rows,)
    num_spec = pl.BlockSpec((block_rows, d), lambda i: (i, 0))
    den_spec = pl.BlockSpec((block_rows, _HD), lambda i: (i, 0))
    w_spec = pl.BlockSpec((d, d), lambda i: (0, 0))
    vec_spec = pl.BlockSpec((1, d), lambda i: (0, 0))
    return pl.pallas_call(
        _out_body,
        grid=grid,
        in_specs=[num_spec, den_spec, w_spec, vec_spec, vec_spec, vec_spec,
                  vec_spec, vec_spec],
        out_specs=pl.BlockSpec((block_rows, d), lambda i: (i, 0)),
        out_shape=jax.ShapeDtypeStruct((n, d), jnp.float32),
    )(num, den, Wout, bout.reshape(1, d), ln1_w.reshape(1, d),
      ln1_b.reshape(1, d), ln2_w.reshape(1, d), ln2_b.reshape(1, d))


def kernel(x, edge_index, WQ, bQ, WK, bK, WV, bV, Wout, bout,
           ln1_w, ln1_b, ln2_w, ln2_b):
    src = edge_index[0, :].astype(jnp.int32)
    dst = edge_index[1, :].astype(jnp.int32)
    n = x.shape[0]
    order = jnp.argsort(src)
    src_s = src[order]
    dst_s = dst[order]
    e = src_s.shape[0]
    npad = n + 8
    epw = e // _NW
    eidx = jnp.arange(e, dtype=jnp.int32)
    core_of_edge = (eidx // epw) // _NS
    dump = core_of_edge * npad + n
    seg_last = jnp.concatenate([src_s[1:] != src_s[:-1],
                                jnp.ones((1,), bool)])
    tile_last = (eidx % epw) == (epw - 1)
    tgt = jnp.where(tile_last | ~seg_last, dump, core_of_edge * npad + src_s)
    srcrep = jnp.broadcast_to(src_s[:, None], (e, _HD))
    qx, kx, vx = _qkv(x, WQ, bQ, WK, bK, WV, bV, block_rows=1000)
    qs, kd, vd = _gather_phase(qx, kx, vx, src_s, dst_s)
    wrows, drows = _edge_math(qs, kd, vd, block_rows=4000)
    pnum, pden, bnum, bden, bidx = _seg_phase(wrows, drows, srcrep, tgt, n)
    num2 = pnum.reshape(_NC, npad, -1)[:, :n]
    den2 = pden.reshape(_NC, npad, -1)[:, :n, :_HD]
    num, den = _merge_phase(num2, den2, bnum, bden, bidx, n)
    return _out_phase(num, den, Wout, bout, ln1_w, ln1_b, ln2_w, ln2_b,
                      block_rows=1000)


# speedup vs baseline: 1.0127x; 1.0127x over previous
import functools

import jax
import jax.numpy as jnp
from jax import lax
from jax.experimental import pallas as pl
from jax.experimental.pallas import tpu as pltpu
from jax.experimental.pallas import tpu_sc as plsc

_NC = 2
_NS = 16
_NW = _NC * _NS
_H = 8
_HD = 16
_BSLOT = 8


def _qkv_body(x_ref, wq_ref, bq_ref, wk_ref, bk_ref, wv_ref, bv_ref,
              q_ref, k_ref, v_ref):
    xb = x_ref[...]
    q_ref[...] = jnp.dot(xb, wq_ref[...], preferred_element_type=jnp.float32) + bq_ref[...]
    k_ref[...] = jnp.dot(xb, wk_ref[...], preferred_element_type=jnp.float32) + bk_ref[...]
    v_ref[...] = jnp.dot(xb, wv_ref[...], preferred_element_type=jnp.float32) + bv_ref[...]


def _qkv(x, WQ, bQ, WK, bK, WV, bV, block_rows):
    n, d = x.shape
    grid = (n // block_rows,)
    row_spec = pl.BlockSpec((block_rows, d), lambda i: (i, 0))
    w_spec = pl.BlockSpec((d, d), lambda i: (0, 0))
    b_spec = pl.BlockSpec((1, d), lambda i: (0, 0))
    out = jax.ShapeDtypeStruct((n, d), jnp.float32)
    return pl.pallas_call(
        _qkv_body,
        grid=grid,
        in_specs=[row_spec, w_spec, b_spec, w_spec, b_spec, w_spec, b_spec],
        out_specs=[row_spec, row_spec, row_spec],
        out_shape=[out, out, out],
    )(x, WQ, bQ.reshape(1, d), WK, bK.reshape(1, d), WV, bV.reshape(1, d))


def _gather_body(n_edges, chunk, qx, kx, vx, src, dst,
                 qs_out, kd_out, vd_out,
                 src_v, dst_v, qb, kb, vb, sem_q, sem_k, sem_v):
    c = lax.axis_index("c")
    s = lax.axis_index("s")
    w = c * _NS + s
    epw = n_edges // _NW
    n_chunks = epw // chunk

    def _chunk_body(ci, _):
        base = w * epw + ci * chunk
        pltpu.sync_copy(src.at[pl.ds(base, chunk)], src_v)
        pltpu.sync_copy(dst.at[pl.ds(base, chunk)], dst_v)
        cp_q = pltpu.async_copy(qx.at[src_v], qb, sem_q)
        cp_k = pltpu.async_copy(kx.at[dst_v], kb, sem_k)
        cp_v = pltpu.async_copy(vx.at[dst_v], vb, sem_v)
        cp_q.wait()
        cp_k.wait()
        cp_v.wait()
        pltpu.sync_copy(qb, qs_out.at[pl.ds(base, chunk)])
        pltpu.sync_copy(kb, kd_out.at[pl.ds(base, chunk)])
        pltpu.sync_copy(vb, vd_out.at[pl.ds(base, chunk)])
        return 0

    lax.fori_loop(0, n_chunks, _chunk_body, 0)


def _gather_phase(qx, kx, vx, src, dst, chunk=200):
    n, d = qx.shape
    e = src.shape[0]
    mesh = plsc.VectorSubcoreMesh(core_axis_name="c", subcore_axis_name="s")
    out = jax.ShapeDtypeStruct((e, d), jnp.float32)
    kfn = pl.kernel(
        functools.partial(_gather_body, e, chunk),
        out_type=(out, out, out),
        mesh=mesh,
        scratch_types=(
            pltpu.VMEM((chunk,), jnp.int32),
            pltpu.VMEM((chunk,), jnp.int32),
            pltpu.VMEM((chunk, d), jnp.float32),
            pltpu.VMEM((chunk, d), jnp.float32),
            pltpu.VMEM((chunk, d), jnp.float32),
            pltpu.SemaphoreType.DMA,
            pltpu.SemaphoreType.DMA,
            pltpu.SemaphoreType.DMA,
        ),
    )
    return kfn(qx, kx, vx, src, dst)


def _edge_math_body(qs_ref, kd_ref, vd_ref, w_ref, den_ref):
    p = qs_ref[...] * kd_ref[...]
    d = p.shape[-1]
    rows = lax.broadcasted_iota(jnp.int32, (d, d), 0)
    cols = lax.broadcasted_iota(jnp.int32, (d, d), 1)
    rep = (rows // _HD == cols // _HD).astype(jnp.float32)
    logits = jnp.dot(p, rep, preferred_element_type=jnp.float32)
    wrep = jnp.exp(logits * jnp.float32(0.25))
    w_ref[...] = wrep * vd_ref[...]
    rows2 = lax.broadcasted_iota(jnp.int32, (d, _HD), 0)
    cols2 = lax.broadcasted_iota(jnp.int32, (d, _HD), 1)
    sel = ((rows2 // _HD == cols2) & (cols2 < _H)).astype(jnp.float32) * jnp.float32(1.0 / _HD)
    den_ref[...] = jnp.dot(wrep, sel, preferred_element_type=jnp.float32)


def _edge_math(qs, kd, vd, block_rows):
    e, d = qs.shape
    grid = (e // block_rows,)
    row_spec = pl.BlockSpec((block_rows, d), lambda i: (i, 0))
    den_spec = pl.BlockSpec((block_rows, _HD), lambda i: (i, 0))
    return pl.pallas_call(
        _edge_math_body,
        grid=grid,
        in_specs=[row_spec, row_spec, row_spec],
        out_specs=[row_spec, den_spec],
        out_shape=[jax.ShapeDtypeStruct((e, d), jnp.float32),
                   jax.ShapeDtypeStruct((e, _HD), jnp.float32)],
    )(qs, kd, vd)


def _seg_body(n_nodes, n_edges, chunk, wrows, drows, srcrep, tgt,
              num_out, den_out, bnum_out, bden_out, bidx_out,
              tgt_v, wb, db, srb, sbuf, sdbuf, bbuf, bdbuf, bibuf,
              sem_t, sem_r, sem_w, sem_d, sem_n, sem_dn):
    c = lax.axis_index("c")
    s = lax.axis_index("s")
    w = c * _NS + s
    epw = n_edges // _NW
    n_chunks = epw // chunk
    npad = n_nodes + 8
    lane = jnp.arange(_HD, dtype=jnp.int32)
    zero_row = jnp.zeros((_HD,), jnp.float32)

    def _zero_bufs(i, _):
        for j in range(_H):
            wb[i, pl.ds(j * _HD, _HD)] = zero_row
            sdbuf[i, pl.ds(j * _HD, _HD)] = zero_row
        db[i, :] = zero_row
        return 0

    lax.fori_loop(0, chunk, _zero_bufs, 0)

    rz = (npad // (8 * _NS)) * 8
    rem = npad - rz * _NS
    row0 = c * npad + s * rz

    def _zero_out(base, nrows):
        off = 0
        while off < nrows:
            step = min(chunk, nrows - off)
            pltpu.sync_copy(wb.at[pl.ds(0, step)], num_out.at[pl.ds(base + off, step)])
            pltpu.sync_copy(sdbuf.at[pl.ds(0, step)], den_out.at[pl.ds(base + off, step)])
            off += step

    _zero_out(row0, rz)

    @pl.when(s == _NS - 1)
    def _():
        _zero_out(c * npad + rz * _NS, rem)

    plsc.subcore_barrier()

    def _chunk_body(ci, carry):
        curv, accs, dacc = carry
        base = w * epw + ci * chunk
        cp_s = pltpu.async_copy(srcrep.at[pl.ds(base, chunk)], srb, sem_r)
        cp_w = pltpu.async_copy(wrows.at[pl.ds(base, chunk)], wb, sem_w)
        cp_d = pltpu.async_copy(drows.at[pl.ds(base, chunk)], db, sem_d)

        @pl.when(ci > 0)
        def _():
            pltpu.make_async_copy(wrows.at[pl.ds(0, chunk)], sbuf, sem_n).wait()
            pltpu.make_async_copy(wrows.at[pl.ds(0, chunk)], sdbuf, sem_dn).wait()

        cp_t = pltpu.async_copy(tgt.at[pl.ds(base, chunk)], tgt_v, sem_t)
        cp_s.wait()
        cp_w.wait()
        cp_d.wait()

        def _edge(e, ecarry):
            curv, accs, dacc = ecarry
            sv = srb[e, :]
            ch = sv != curv
            new_accs = []
            for h in range(_H):
                wv = wb[e, pl.ds(h * _HD, _HD)]
                na = jnp.where(ch, wv, accs[h] + wv)
                sbuf[e, pl.ds(h * _HD, _HD)] = na
                new_accs.append(na)
            dv = db[e, :]
            dacc = jnp.where(ch, dv, dacc + dv)
            sdbuf[e, pl.ds(0, _HD)] = dacc
            return sv, tuple(new_accs), dacc

        curv, accs, dacc = lax.fori_loop(0, chunk, _edge, (curv, accs, dacc))

        cp_t.wait()
        pltpu.async_copy(sbuf, num_out.at[tgt_v], sem_n)
        pltpu.async_copy(sdbuf, den_out.at[tgt_v], sem_dn)
        return curv, accs, dacc

    acc0 = tuple(jnp.zeros((_HD,), jnp.float32) for _ in range(_H))
    curv, accs, dacc = lax.fori_loop(
        0, n_chunks, _chunk_body,
        (jnp.full((_HD,), -1, jnp.int32), acc0, jnp.zeros((_HD,), jnp.float32)))

    pltpu.make_async_copy(wrows.at[pl.ds(0, chunk)], sbuf, sem_n).wait()
    pltpu.make_async_copy(wrows.at[pl.ds(0, chunk)], sdbuf, sem_dn).wait()

    for h in range(_H):
        bbuf[1, pl.ds(h * _HD, _HD)] = accs[h]
        bbuf[0, pl.ds(h * _HD, _HD)] = zero_row
    bdbuf[1, :] = dacc
    bdbuf[0, :] = zero_row
    bibuf[...] = jnp.where(lane == 1, curv, jnp.full((_HD,), -1, jnp.int32))

    pltpu.sync_copy(bbuf, bnum_out.at[pl.ds(w * _BSLOT, _BSLOT)])
    pltpu.sync_copy(bdbuf, bden_out.at[pl.ds(w * _BSLOT, _BSLOT)])
    pltpu.sync_copy(bibuf.at[pl.ds(0, _BSLOT)], bidx_out.at[pl.ds(w * _BSLOT, _BSLOT)])


def _seg_phase(wrows, drows, srcrep, tgt, n_nodes, chunk=200):
    e, d = wrows.shape
    npad = n_nodes + 8
    mesh = plsc.VectorSubcoreMesh(core_axis_name="c", subcore_axis_name="s")
    kfn = pl.kernel(
        functools.partial(_seg_body, n_nodes, e, chunk),
        out_type=(
            jax.ShapeDtypeStruct((_NC * npad, d), jnp.float32),
            jax.ShapeDtypeStruct((_NC * npad, d), jnp.float32),
            jax.ShapeDtypeStruct((_NW * _BSLOT, d), jnp.float32),
            jax.ShapeDtypeStruct((_NW * _BSLOT, _HD), jnp.float32),
            jax.ShapeDtypeStruct((_NW * _BSLOT,), jnp.int32),
        ),
        mesh=mesh,
        scratch_types=(
            pltpu.VMEM((chunk,), jnp.int32),
            pltpu.VMEM((chunk, d), jnp.float32),
            pltpu.VMEM((chunk, _HD), jnp.float32),
            pltpu.VMEM((chunk, _HD), jnp.int32),
            pltpu.VMEM((chunk, d), jnp.float32),
            pltpu.VMEM((chunk, d), jnp.float32),
            pltpu.VMEM((_BSLOT, d), jnp.float32),
            pltpu.VMEM((_BSLOT, _HD), jnp.float32),
            pltpu.VMEM((_HD,), jnp.int32),
            pltpu.SemaphoreType.DMA,
            pltpu.SemaphoreType.DMA,
            pltpu.SemaphoreType.DMA,
            pltpu.SemaphoreType.DMA,
            pltpu.SemaphoreType.DMA,
            pltpu.SemaphoreType.DMA,
        ),
    )
    return kfn(wrows, drows, srcrep, tgt)


def _merge_body(num_ref, den_ref, bnum_ref, bden_ref, bidx_ref,
                onum_ref, oden_ref):
    onum_ref[...] = num_ref[0] + num_ref[1]
    oden_ref[...] = den_ref[0] + den_ref[1]

    def _apply(i, _):
        idx = bidx_ref[i]

        @pl.when(idx >= 0)
        def _():
            onum_ref[pl.ds(idx, 1), :] = onum_ref[pl.ds(idx, 1), :] + bnum_ref[pl.ds(i, 1), :]
            oden_ref[pl.ds(idx, 1), :] = oden_ref[pl.ds(idx, 1), :] + bden_ref[pl.ds(i, 1), :]

        return 0

    lax.fori_loop(0, _NW * _BSLOT, _apply, 0)


def _merge_phase(num2, den2, bnum, bden, bidx, n_nodes):
    d = num2.shape[-1]
    nb = _NW * _BSLOT
    return pl.pallas_call(
        _merge_body,
        in_specs=[
            pl.BlockSpec((_NC, n_nodes, d), lambda: (0, 0, 0)),
            pl.BlockSpec((_NC, n_nodes, _HD), lambda: (0, 0, 0)),
            pl.BlockSpec((nb, d), lambda: (0, 0)),
            pl.BlockSpec((nb, _HD), lambda: (0, 0)),
            pl.BlockSpec(memory_space=pltpu.SMEM),
        ],
        out_specs=[pl.BlockSpec((n_nodes, d), lambda: (0, 0)),
                   pl.BlockSpec((n_nodes, _HD), lambda: (0, 0))],
        out_shape=[jax.ShapeDtypeStruct((n_nodes, d), jnp.float32),
                   jax.ShapeDtypeStruct((n_nodes, _HD), jnp.float32)],
    )(num2, den2, bnum, bden, bidx)


def _out_body(num_ref, den_ref, wout_ref, bout_ref,
              ln1w_ref, ln1b_ref, ln2w_ref, ln2b_ref, o_ref):
    nb = num_ref[...]
    db = den_ref[...]
    d = nb.shape[-1]
    rows = lax.broadcasted_iota(jnp.int32, (_HD, d), 0)
    cols = lax.broadcasted_iota(jnp.int32, (_HD, d), 1)
    rep_m = (cols // _HD == rows).astype(jnp.float32)
    rep = jnp.dot(db, rep_m, preferred_element_type=jnp.float32)
    attn = nb / (rep + jnp.float32(1e-16))
    mu = jnp.mean(attn, axis=1, keepdims=True)
    xc = attn - mu
    var = jnp.mean(xc * xc, axis=1, keepdims=True)
    ln1 = xc / jnp.sqrt(var + jnp.float32(1e-5)) * ln1w_ref[...] + ln1b_ref[...]
    o = jnp.dot(ln1, wout_ref[...], preferred_element_type=jnp.float32) + bout_ref[...]
    mu2 = jnp.mean(o, axis=1, keepdims=True)
    oc = o - mu2
    var2 = jnp.mean(oc * oc, axis=1, keepdims=True)
    o_ref[...] = oc / jnp.sqrt(var2 + jnp.float32(1e-5)) * ln2w_ref[...] + ln2b_ref[...]


def _out_phase(num, den, Wout, bout, ln1_w, ln1_b, ln2_w, ln2_b, block_rows):
    n, d = num.shape
    grid = (n // block_rows,)
    num_spec = pl.BlockSpec((block_rows, d), lambda i: (i, 0))
    den_spec = pl.BlockSpec((block_rows, _HD), lambda i: (i, 0))
    w_spec = pl.BlockSpec((d, d), lambda i: (0, 0))
    vec_spec = pl.BlockSpec((1, d), lambda i: (0, 0))
    return pl.pallas_call(
        _out_body,
        grid=grid,
        in_specs=[num_spec, den_spec, w_spec, vec_spec, vec_spec, vec_spec,
                  vec_spec, vec_spec],
        out_specs=pl.BlockSpec((block_rows, d), lambda i: (i, 0)),
        out_shape=jax.ShapeDtypeStruct((n, d), jnp.float32),
    )(num, den, Wout, bout.reshape(1, d), ln1_w.reshape(1, d),
      ln1_b.reshape(1, d), ln2_w.reshape(1, d), ln2_b.reshape(1, d))


def kernel(x, edge_index, WQ, bQ, WK, bK, WV, bV, Wout, bout,
           ln1_w, ln1_b, ln2_w, ln2_b):
    src = edge_index[0, :].astype(jnp.int32)
    dst = edge_index[1, :].astype(jnp.int32)
    n = x.shape[0]
    order = jnp.argsort(src)
    src_s = src[order]
    dst_s = dst[order]
    e = src_s.shape[0]
    npad = n + 8
    epw = e // _NW
    eidx = jnp.arange(e, dtype=jnp.int32)
    core_of_edge = (eidx // epw) // _NS
    dump = core_of_edge * npad + n
    seg_last = jnp.concatenate([src_s[1:] != src_s[:-1],
                                jnp.ones((1,), bool)])
    tile_last = (eidx % epw) == (epw - 1)
    tgt = jnp.where(tile_last | ~seg_last, dump, core_of_edge * npad + src_s)
    srcrep = jnp.broadcast_to(src_s[:, None], (e, _HD))
    qx, kx, vx = _qkv(x, WQ, bQ, WK, bK, WV, bV, block_rows=1000)
    qs, kd, vd = _gather_phase(qx, kx, vx, src_s, dst_s)
    wrows, drows = _edge_math(qs, kd, vd, block_rows=4000)
    pnum, pden, bnum, bden, bidx = _seg_phase(wrows, drows, srcrep, tgt, n)
    num2 = pnum.reshape(_NC, npad, -1)[:, :n]
    den2 = pden.reshape(_NC, npad, -1)[:, :n, :_HD]
    num, den = _merge_phase(num2, den2, bnum, bden, bidx, n)
    return _out_phase(num, den, Wout, bout, ln1_w, ln1_b, ln2_w, ln2_b,
                      block_rows=1000)
